# Initial kernel scaffold; baseline (speedup 1.0000x reference)
import jax
import jax.numpy as jnp
from jax.experimental import pallas as pl


def kernel(logits, labels):
    raise NotImplementedError("write your pallas kernel here")


# speedup vs baseline: 5.1163x; 5.1163x over previous
import functools

import jax
import jax.numpy as jnp
from jax import lax
from jax.experimental import pallas as pl
from jax.experimental.pallas import tpu as pltpu
from jax.experimental.pallas import tpu_sc as plsc

_THRESH = 0.35667494393873245
_LB_IGNORE = 255
_FACTOR = 16

_NB, _NC, _H, _W = 8, 19, 512, 512
_PIX_PER_IMG = _H * _W
_NPIX = _NB * _PIX_PER_IMG
_NWORK = 32
_PIX_PER_WORK = _NPIX // _NWORK
_CHUNK = 2048
_NCHUNK = _PIX_PER_WORK // _CHUNK
_GROUPS = _CHUNK // 16
_QUARTERS = _PIX_PER_IMG // _PIX_PER_WORK

_LN2 = 0.6931471805599453
_K_STATIC = max(_NPIX // _FACTOR, 1)


def _log_f32(s):
    bits = lax.bitcast_convert_type(s, jnp.int32)
    e = jnp.right_shift(bits, 23) - 127
    m = lax.bitcast_convert_type(
        jnp.bitwise_or(jnp.bitwise_and(bits, 0x7FFFFF), 0x3F800000),
        jnp.float32)
    big = m > 1.4142135623730951
    m = jnp.where(big, m * 0.5, m)
    ef = (e + jnp.where(big, 1, 0)).astype(jnp.float32)
    t = (m - 1.0) / (m + 1.0)
    u = t * t
    p = t * (2.0 + u * (0.6666666666 + u * (0.4 + u * 0.2857142857)))
    return ef * _LN2 + p


def _sc_body(logits_hbm, labels_hbm, loss_hbm, cnt_out, sum_out, val_out,
             lbuf, labv, lossv, stage, sem, osem):
    cid = lax.axis_index("c")
    sid = lax.axis_index("s")
    wid = sid * 2 + cid
    img = wid // _QUARTERS
    quarter = wid % _QUARTERS
    pix0 = img * _PIX_PER_IMG + quarter * _PIX_PER_WORK
    off0 = quarter * _PIX_PER_WORK

    zeros = jnp.zeros((16,), jnp.float32)
    iota16 = lax.iota(jnp.int32, 16)

    def chunk_body(j, carry):
        off = off0 + j * _CHUNK
        base = pix0 + j * _CHUNK
        copies = []
        for c in range(_NC):
            src = logits_hbm.at[pl.ds((img * _NC + c) * _PIX_PER_IMG + off,
                                      _CHUNK)]
            copies.append(
                pltpu.async_copy(src, lbuf.at[pl.ds(c * _CHUNK, _CHUNK)], sem))
        labcp = pltpu.async_copy(labels_hbm.at[pl.ds(base, _CHUNK)], labv, sem)
        for cp in copies:
            cp.wait()
        labcp.wait()

        def group_body(g, acc):
            cnt_h, sum_h, cnt_v = acc
            gs = pl.multiple_of(g * 16, 16)
            z = [lbuf[pl.ds(c * _CHUNK + gs, 16)] for c in range(_NC)]
            m = z[0]
            for c in range(1, _NC):
                m = jnp.maximum(m, z[c])
            s = jnp.exp(z[0] - m)
            for c in range(1, _NC):
                s = s + jnp.exp(z[c] - m)
            lab = labv[pl.ds(gs, 16)]
            valid = lab != _LB_IGNORE
            labc = jnp.minimum(jnp.maximum(lab, 0), _NC - 1)
            z_l = z[0]
            for c in range(1, _NC):
                z_l = jnp.where(labc == c, z[c], z_l)
            loss = _log_f32(s) - (z_l - m)
            loss = jnp.where(valid, loss, 0.0)
            hard = loss > _THRESH
            cnt_h = cnt_h + jnp.where(hard, 1.0, 0.0)
            sum_h = sum_h + jnp.where(hard, loss, 0.0)
            cnt_v = cnt_v + jnp.where(valid, 1.0, 0.0)
            lossv[pl.ds(gs, 16)] = loss
            return (cnt_h, sum_h, cnt_v)

        carry = lax.fori_loop(0, _GROUPS, group_body, carry)
        pltpu.sync_copy(lossv, loss_hbm.at[pl.ds(base, _CHUNK)])
        return carry

    cnt_h, sum_h, cnt_v = lax.fori_loop(
        0, _NCHUNK, chunk_body, (zeros, zeros, zeros))

    stage[pl.ds(0, 16)] = cnt_h
    pltpu.sync_copy(stage, cnt_out.at[pl.ds(wid * 16, 16)])
    stage[pl.ds(0, 16)] = sum_h
    pltpu.sync_copy(stage, sum_out.at[pl.ds(wid * 16, 16)])
    stage[pl.ds(0, 16)] = cnt_v
    pltpu.sync_copy(stage, val_out.at[pl.ds(wid * 16, 16)])


def _sc_main(logits_flat, labels_flat):
    mesh = plsc.VectorSubcoreMesh(core_axis_name="c", subcore_axis_name="s")
    f = pl.kernel(
        _sc_body,
        mesh=mesh,
        out_type=[
            jax.ShapeDtypeStruct((_NPIX,), jnp.float32),
            jax.ShapeDtypeStruct((_NWORK * 16,), jnp.float32),
            jax.ShapeDtypeStruct((_NWORK * 16,), jnp.float32),
            jax.ShapeDtypeStruct((_NWORK * 16,), jnp.float32),
        ],
        scratch_types=[
            pltpu.VMEM((_NC * _CHUNK,), jnp.float32),
            pltpu.VMEM((_CHUNK,), jnp.int32),
            pltpu.VMEM((_CHUNK,), jnp.float32),
            pltpu.VMEM((16,), jnp.float32),
            pltpu.SemaphoreType.DMA,
            pltpu.SemaphoreType.DMA,
        ],
    )
    return f(logits_flat, labels_flat)


def _topk_mean_body(x_ref, o_ref):
    x = jnp.maximum(x_ref[...], 0.0)
    bits = lax.bitcast_convert_type(x, jnp.int32)
    kf = jnp.float32(_K_STATIC)

    def step(i, cand):
        test = jnp.bitwise_or(cand, lax.shift_left(jnp.int32(1), 30 - i))
        cnt = jnp.sum((bits >= test).astype(jnp.float32))
        return jnp.where(cnt >= kf, test, cand)

    cand = lax.fori_loop(0, 31, step, jnp.int32(0))
    v = lax.bitcast_convert_type(cand, jnp.float32)
    gt = x > v
    cnt_gt = jnp.sum(gt.astype(jnp.float32))
    sum_gt = jnp.sum(jnp.where(gt, x, 0.0))
    o_ref[0, 0] = (sum_gt + (kf - cnt_gt) * v) / kf


def _topk_mean(loss_flat):
    x = loss_flat.reshape(_NPIX // 128, 128)
    out = pl.pallas_call(
        _topk_mean_body,
        out_shape=jax.ShapeDtypeStruct((1, 1), jnp.float32),
        out_specs=pl.BlockSpec(memory_space=pltpu.SMEM),
    )(x)
    return out[0, 0]


def kernel(logits, labels):
    logits_flat = logits.reshape(-1)
    labels_flat = labels.reshape(-1)
    loss, cnt_h, sum_h, cnt_v = _sc_main(logits_flat, labels_flat)
    n_hard = jnp.sum(cnt_h).astype(jnp.int32)
    n_valid = jnp.sum(cnt_v).astype(jnp.int32)
    sum_hard = jnp.sum(sum_h)
    n_min = n_valid // _FACTOR
    mean_hard = sum_hard / jnp.maximum(n_hard, 1).astype(jnp.float32)
    pred = n_hard < n_min
    return lax.cond(pred, _topk_mean, lambda _: mean_hard, loss)


# speedup vs baseline: 6.1622x; 1.2044x over previous
import functools

import jax
import jax.numpy as jnp
from jax import lax
from jax.experimental import pallas as pl
from jax.experimental.pallas import tpu as pltpu
from jax.experimental.pallas import tpu_sc as plsc

_THRESH = 0.35667494393873245
_LB_IGNORE = 255
_FACTOR = 16

_NB, _NC, _H, _W = 8, 19, 512, 512
_PIX_PER_IMG = _H * _W
_NPIX = _NB * _PIX_PER_IMG
_NWORK = 32
_PIX_PER_WORK = _NPIX // _NWORK
_CHUNK = 2048
_NCHUNK = _PIX_PER_WORK // _CHUNK
_GROUPS = _CHUNK // 16
_QUARTERS = _PIX_PER_IMG // _PIX_PER_WORK

_LN2 = 0.6931471805599453
_K_STATIC = max(_NPIX // _FACTOR, 1)


def _log_f32(s):
    bits = lax.bitcast_convert_type(s, jnp.int32)
    e = jnp.right_shift(bits, 23) - 127
    m = lax.bitcast_convert_type(
        jnp.bitwise_or(jnp.bitwise_and(bits, 0x7FFFFF), 0x3F800000),
        jnp.float32)
    big = m > 1.4142135623730951
    m = jnp.where(big, m * 0.5, m)
    ef = (e + jnp.where(big, 1, 0)).astype(jnp.float32)
    t = (m - 1.0) / (m + 1.0)
    u = t * t
    p = t * (2.0 + u * (0.6666666666 + u * (0.4 + u * 0.2857142857)))
    return ef * _LN2 + p


def _tree(xs, op):
    xs = list(xs)
    while len(xs) > 1:
        nxt = [op(xs[i], xs[i + 1]) for i in range(0, len(xs) - 1, 2)]
        if len(xs) % 2:
            nxt.append(xs[-1])
        xs = nxt
    return xs[0]


def _sc_body(logits_hbm, labels_hbm, loss_hbm, cnt_out, sum_out, val_out,
             lbuf, labv, lossv, stage, sem, osem):
    cid = lax.axis_index("c")
    sid = lax.axis_index("s")
    wid = sid * 2 + cid
    img = wid // _QUARTERS
    quarter = wid % _QUARTERS
    pix0 = img * _PIX_PER_IMG + quarter * _PIX_PER_WORK
    off0 = quarter * _PIX_PER_WORK

    zeros = jnp.zeros((16,), jnp.float32)

    def fire_chunk(k, slot):
        off = off0 + k * _CHUNK
        base = pix0 + k * _CHUNK
        for c in range(_NC):
            src = logits_hbm.at[pl.ds((img * _NC + c) * _PIX_PER_IMG + off,
                                      _CHUNK)]
            pltpu.async_copy(
                src, lbuf.at[pl.ds((slot * _NC + c) * _CHUNK, _CHUNK)], sem)
        pltpu.async_copy(labels_hbm.at[pl.ds(base, _CHUNK)],
                         labv.at[pl.ds(slot * _CHUNK, _CHUNK)], sem)

    def drain_chunk(slot):
        for c in range(_NC):
            pltpu.make_async_copy(
                logits_hbm.at[pl.ds(0, _CHUNK)],
                lbuf.at[pl.ds((slot * _NC + c) * _CHUNK, _CHUNK)], sem).wait()
        pltpu.make_async_copy(labels_hbm.at[pl.ds(0, _CHUNK)],
                              labv.at[pl.ds(slot * _CHUNK, _CHUNK)],
                              sem).wait()

    def compute_chunk(k, slot, carry):
        lb = slot * _NC * _CHUNK
        sb = slot * _CHUNK

        def group_body(g, acc):
            cnt_h, sum_h, cnt_v = acc
            gs = pl.multiple_of(g * 16, 16)
            z = [lbuf[pl.ds(lb + c * _CHUNK + gs, 16)] for c in range(_NC)]
            m = _tree(z, jnp.maximum)
            s = _tree([jnp.exp(zc - m) for zc in z], jnp.add)
            lab = labv[pl.ds(sb + gs, 16)]
            valid = lab != _LB_IGNORE
            labc = jnp.minimum(jnp.maximum(lab, 0), _NC - 1)
            z_l = z[0]
            for c in range(1, _NC):
                z_l = jnp.where(labc == c, z[c], z_l)
            loss = _log_f32(s) - (z_l - m)
            loss = jnp.where(valid, loss, 0.0)
            hard = loss > _THRESH
            cnt_h = cnt_h + jnp.where(hard, 1.0, 0.0)
            sum_h = sum_h + jnp.where(hard, loss, 0.0)
            cnt_v = cnt_v + jnp.where(valid, 1.0, 0.0)
            lossv[pl.ds(sb + gs, 16)] = loss
            return (cnt_h, sum_h, cnt_v)

        carry = lax.fori_loop(0, _GROUPS, group_body, carry)
        base = pix0 + k * _CHUNK
        pltpu.sync_copy(lossv.at[pl.ds(sb, _CHUNK)],
                        loss_hbm.at[pl.ds(base, _CHUNK)])
        return carry

    fire_chunk(0, 0)
    fire_chunk(1, 1)

    def pair_body(j2, carry):
        for slot in (0, 1):
            k = j2 * 2 + slot
            drain_chunk(slot)
            carry = compute_chunk(k, slot, carry)

            @pl.when(k + 2 < _NCHUNK)
            def _():
                fire_chunk(k + 2, slot)
        return carry

    cnt_h, sum_h, cnt_v = lax.fori_loop(
        0, _NCHUNK // 2, pair_body, (zeros, zeros, zeros))

    stage[pl.ds(0, 16)] = cnt_h
    pltpu.sync_copy(stage, cnt_out.at[pl.ds(wid * 16, 16)])
    stage[pl.ds(0, 16)] = sum_h
    pltpu.sync_copy(stage, sum_out.at[pl.ds(wid * 16, 16)])
    stage[pl.ds(0, 16)] = cnt_v
    pltpu.sync_copy(stage, val_out.at[pl.ds(wid * 16, 16)])


def _sc_main(logits_flat, labels_flat):
    mesh = plsc.VectorSubcoreMesh(core_axis_name="c", subcore_axis_name="s")
    f = pl.kernel(
        _sc_body,
        mesh=mesh,
        out_type=[
            jax.ShapeDtypeStruct((_NPIX,), jnp.float32),
            jax.ShapeDtypeStruct((_NWORK * 16,), jnp.float32),
            jax.ShapeDtypeStruct((_NWORK * 16,), jnp.float32),
            jax.ShapeDtypeStruct((_NWORK * 16,), jnp.float32),
        ],
        scratch_types=[
            pltpu.VMEM((2 * _NC * _CHUNK,), jnp.float32),
            pltpu.VMEM((2 * _CHUNK,), jnp.int32),
            pltpu.VMEM((2 * _CHUNK,), jnp.float32),
            pltpu.VMEM((16,), jnp.float32),
            pltpu.SemaphoreType.DMA,
            pltpu.SemaphoreType.DMA,
        ],
    )
    return f(logits_flat, labels_flat)


def _topk_mean_body(x_ref, o_ref):
    x = jnp.maximum(x_ref[...], 0.0)
    bits = lax.bitcast_convert_type(x, jnp.int32)
    kf = jnp.float32(_K_STATIC)

    def step(i, cand):
        test = jnp.bitwise_or(cand, lax.shift_left(jnp.int32(1), 30 - i))
        cnt = jnp.sum((bits >= test).astype(jnp.float32))
        return jnp.where(cnt >= kf, test, cand)

    cand = lax.fori_loop(0, 31, step, jnp.int32(0))
    v = lax.bitcast_convert_type(cand, jnp.float32)
    gt = x > v
    cnt_gt = jnp.sum(gt.astype(jnp.float32))
    sum_gt = jnp.sum(jnp.where(gt, x, 0.0))
    o_ref[0, 0] = (sum_gt + (kf - cnt_gt) * v) / kf


def _topk_mean(loss_flat):
    x = loss_flat.reshape(_NPIX // 128, 128)
    out = pl.pallas_call(
        _topk_mean_body,
        out_shape=jax.ShapeDtypeStruct((1, 1), jnp.float32),
        out_specs=pl.BlockSpec(memory_space=pltpu.SMEM),
    )(x)
    return out[0, 0]


def kernel(logits, labels):
    logits_flat = logits.reshape(-1)
    labels_flat = labels.reshape(-1)
    loss, cnt_h, sum_h, cnt_v = _sc_main(logits_flat, labels_flat)
    n_hard = jnp.sum(cnt_h).astype(jnp.int32)
    n_valid = jnp.sum(cnt_v).astype(jnp.int32)
    sum_hard = jnp.sum(sum_h)
    n_min = n_valid // _FACTOR
    mean_hard = sum_hard / jnp.maximum(n_hard, 1).astype(jnp.float32)
    pred = n_hard < n_min
    return lax.cond(pred, _topk_mean, lambda _: mean_hard, loss)


# speedup vs baseline: 8.7290x; 1.4165x over previous
import functools

import jax
import jax.numpy as jnp
from jax import lax
from jax.experimental import pallas as pl
from jax.experimental.pallas import tpu as pltpu
from jax.experimental.pallas import tpu_sc as plsc

_THRESH = 0.35667494393873245
_LB_IGNORE = 255
_FACTOR = 16

_NB, _NC, _H, _W = 8, 19, 512, 512
_NPIX = _NB * _H * _W
_NWORK = 32
_ROWS_PER_WORK = _H // 4
_CR = 8
_CW = 256
_CHUNK = _CR * _CW
_NSTRIPE = _ROWS_PER_WORK // _CR
_NHALF = _W // _CW
_NCHUNK = _NSTRIPE * _NHALF
_GROUPS = _CHUNK // 16
_GROUPS_PER_ROW = _CW // 16

_LN2 = 0.6931471805599453
_K_STATIC = max(_NPIX // _FACTOR, 1)


def _log_f32(s):
    bits = lax.bitcast_convert_type(s, jnp.int32)
    e = jnp.right_shift(bits, 23) - 127
    m = lax.bitcast_convert_type(
        jnp.bitwise_or(jnp.bitwise_and(bits, 0x7FFFFF), 0x3F800000),
        jnp.float32)
    big = m > 1.4142135623730951
    m = jnp.where(big, m * 0.5, m)
    ef = (e + jnp.where(big, 1, 0)).astype(jnp.float32)
    t = (m - 1.0) / (m + 1.0)
    u = t * t
    p = t * (2.0 + u * (0.6666666666 + u * (0.4 + u * 0.2857142857)))
    return ef * _LN2 + p


def _tree(xs, op):
    xs = list(xs)
    while len(xs) > 1:
        nxt = [op(xs[i], xs[i + 1]) for i in range(0, len(xs) - 1, 2)]
        if len(xs) % 2:
            nxt.append(xs[-1])
        xs = nxt
    return xs[0]


def _sc_body(logits_hbm, labels_hbm, loss_hbm, cnt_out, sum_out, val_out,
             lbuf, labv, lossv, stage, sem, osem):
    cid = lax.axis_index("c")
    sid = lax.axis_index("s")
    wid = sid * 2 + cid
    img = wid // 4
    quarter = wid % 4
    row0 = quarter * _ROWS_PER_WORK

    zeros = jnp.zeros((16,), jnp.float32)

    def chunk_coords(k):
        stripe = k // _NHALF
        half = k % _NHALF
        return row0 + stripe * _CR, half * _CW

    def fire_chunk(k, slot):
        r0, w0 = chunk_coords(k)
        for c in range(_NC):
            src = logits_hbm.at[img, c, pl.ds(r0, _CR), pl.ds(w0, _CW)]
            pltpu.async_copy(
                src, lbuf.at[pl.ds((slot * _NC + c) * _CR, _CR), :], sem)
        pltpu.async_copy(labels_hbm.at[img, pl.ds(r0, _CR), pl.ds(w0, _CW)],
                         labv.at[pl.ds(slot * _CR, _CR), :], sem)

    def drain_chunk(slot):
        for c in range(_NC):
            pltpu.make_async_copy(
                logits_hbm.at[0, 0, pl.ds(0, _CR), pl.ds(0, _CW)],
                lbuf.at[pl.ds((slot * _NC + c) * _CR, _CR), :], sem).wait()
        pltpu.make_async_copy(
            labels_hbm.at[0, pl.ds(0, _CR), pl.ds(0, _CW)],
            labv.at[pl.ds(slot * _CR, _CR), :], sem).wait()

    def compute_chunk(k, slot, carry):
        def group_body(g, acc):
            cnt_h, sum_h, cnt_v = acc
            r = jnp.right_shift(g, 4)
            w = pl.multiple_of(jnp.bitwise_and(g, 15) * 16, 16)
            z = [lbuf[(slot * _NC + c) * _CR + r, pl.ds(w, 16)]
                 for c in range(_NC)]
            m = _tree(z, jnp.maximum)
            s = _tree([jnp.exp(zc - m) for zc in z], jnp.add)
            lab = labv[slot * _CR + r, pl.ds(w, 16)]
            valid = lab != _LB_IGNORE
            labc = jnp.minimum(jnp.maximum(lab, 0), _NC - 1)
            z_l = z[0]
            for c in range(1, _NC):
                z_l = jnp.where(labc == c, z[c], z_l)
            loss = _log_f32(s) - (z_l - m)
            loss = jnp.where(valid, loss, 0.0)
            hard = loss > _THRESH
            cnt_h = cnt_h + jnp.where(hard, 1.0, 0.0)
            sum_h = sum_h + jnp.where(hard, loss, 0.0)
            cnt_v = cnt_v + jnp.where(valid, 1.0, 0.0)
            lossv[slot * _CR + r, pl.ds(w, 16)] = loss
            return (cnt_h, sum_h, cnt_v)

        carry = lax.fori_loop(0, _GROUPS, group_body, carry)
        r0, w0 = chunk_coords(k)
        pltpu.sync_copy(lossv.at[pl.ds(slot * _CR, _CR), :],
                        loss_hbm.at[img, pl.ds(r0, _CR), pl.ds(w0, _CW)])
        return carry

    fire_chunk(0, 0)
    fire_chunk(1, 1)

    def pair_body(j2, carry):
        for slot in (0, 1):
            k = j2 * 2 + slot
            drain_chunk(slot)
            carry = compute_chunk(k, slot, carry)

            @pl.when(k + 2 < _NCHUNK)
            def _():
                fire_chunk(k + 2, slot)
        return carry

    cnt_h, sum_h, cnt_v = lax.fori_loop(
        0, _NCHUNK // 2, pair_body, (zeros, zeros, zeros))

    stage[pl.ds(0, 16)] = cnt_h
    pltpu.sync_copy(stage, cnt_out.at[pl.ds(wid * 16, 16)])
    stage[pl.ds(0, 16)] = sum_h
    pltpu.sync_copy(stage, sum_out.at[pl.ds(wid * 16, 16)])
    stage[pl.ds(0, 16)] = cnt_v
    pltpu.sync_copy(stage, val_out.at[pl.ds(wid * 16, 16)])


def _sc_main(logits, labels):
    mesh = plsc.VectorSubcoreMesh(core_axis_name="c", subcore_axis_name="s")
    f = pl.kernel(
        _sc_body,
        mesh=mesh,
        out_type=[
            jax.ShapeDtypeStruct((_NB, _H, _W), jnp.float32),
            jax.ShapeDtypeStruct((_NWORK * 16,), jnp.float32),
            jax.ShapeDtypeStruct((_NWORK * 16,), jnp.float32),
            jax.ShapeDtypeStruct((_NWORK * 16,), jnp.float32),
        ],
        scratch_types=[
            pltpu.VMEM((2 * _NC * _CR, _CW), jnp.float32),
            pltpu.VMEM((2 * _CR, _CW), jnp.int32),
            pltpu.VMEM((2 * _CR, _CW), jnp.float32),
            pltpu.VMEM((16,), jnp.float32),
            pltpu.SemaphoreType.DMA,
            pltpu.SemaphoreType.DMA,
        ],
    )
    return f(logits, labels)


def _topk_mean_body(x_ref, o_ref):
    x = jnp.maximum(x_ref[...], 0.0)
    bits = lax.bitcast_convert_type(x, jnp.int32)
    kf = jnp.float32(_K_STATIC)

    def step(i, cand):
        test = jnp.bitwise_or(cand, lax.shift_left(jnp.int32(1), 30 - i))
        cnt = jnp.sum((bits >= test).astype(jnp.float32))
        return jnp.where(cnt >= kf, test, cand)

    cand = lax.fori_loop(0, 31, step, jnp.int32(0))
    v = lax.bitcast_convert_type(cand, jnp.float32)
    gt = x > v
    cnt_gt = jnp.sum(gt.astype(jnp.float32))
    sum_gt = jnp.sum(jnp.where(gt, x, 0.0))
    o_ref[0, 0] = (sum_gt + (kf - cnt_gt) * v) / kf


def _topk_mean(loss):
    out = pl.pallas_call(
        _topk_mean_body,
        out_shape=jax.ShapeDtypeStruct((1, 1), jnp.float32),
        out_specs=pl.BlockSpec(memory_space=pltpu.SMEM),
    )(loss)
    return out[0, 0]


def kernel(logits, labels):
    loss, cnt_h, sum_h, cnt_v = _sc_main(logits, labels)
    n_hard = jnp.sum(cnt_h).astype(jnp.int32)
    n_valid = jnp.sum(cnt_v).astype(jnp.int32)
    sum_hard = jnp.sum(sum_h)
    n_min = n_valid // _FACTOR
    mean_hard = sum_hard / jnp.maximum(n_hard, 1).astype(jnp.float32)
    pred = n_hard < n_min
    return lax.cond(pred, _topk_mean, lambda _: mean_hard, loss)


# speedup vs baseline: 9.4585x; 1.0836x over previous
import functools

import jax
import jax.numpy as jnp
from jax import lax
from jax.experimental import pallas as pl
from jax.experimental.pallas import tpu as pltpu
from jax.experimental.pallas import tpu_sc as plsc

_THRESH = 0.35667494393873245
_LB_IGNORE = 255
_FACTOR = 16

_NB, _NC, _H, _W = 8, 19, 512, 512
_NPIX = _NB * _H * _W
_NWORK = 32
_ROWS_PER_WORK = _H // 4
_CR = 8
_CW = 256
_CHUNK = _CR * _CW
_NSTRIPE = _ROWS_PER_WORK // _CR
_NHALF = _W // _CW
_NCHUNK = _NSTRIPE * _NHALF
_GROUPS = _CHUNK // 16
_GROUPS_PER_ROW = _CW // 16

_LN2 = 0.6931471805599453
_K_STATIC = max(_NPIX // _FACTOR, 1)


def _log_f32(s):
    bits = lax.bitcast_convert_type(s, jnp.int32)
    e = jnp.right_shift(bits, 23) - 127
    m = lax.bitcast_convert_type(
        jnp.bitwise_or(jnp.bitwise_and(bits, 0x7FFFFF), 0x3F800000),
        jnp.float32)
    big = m > 1.4142135623730951
    m = jnp.where(big, m * 0.5, m)
    ef = (e + jnp.where(big, 1, 0)).astype(jnp.float32)
    z = m - 1.0
    c8, c7, c6, c5, c4, c3, c2, c1, c0 = (
        7.0376836292e-2, -1.1514610310e-1, 1.1676998740e-1,
        -1.2420140846e-1, 1.4249322787e-1, -1.6668057665e-1,
        2.0000714765e-1, -2.4999993993e-1, 3.3333331174e-1)
    z2 = z * z
    z4 = z2 * z2
    b0 = c1 * z + c0
    b1 = c3 * z + c2
    b2 = c5 * z + c4
    b3 = c7 * z + c6
    d0 = b1 * z2 + b0
    d1 = b3 * z2 + b2
    poly = (c8 * z4 + d1) * z4 + d0
    r = z * z2 * poly - 0.5 * z2
    return z + r + ef * _LN2


def _tree(xs, op):
    xs = list(xs)
    while len(xs) > 1:
        nxt = [op(xs[i], xs[i + 1]) for i in range(0, len(xs) - 1, 2)]
        if len(xs) % 2:
            nxt.append(xs[-1])
        xs = nxt
    return xs[0]


def _sc_body(logits_hbm, labels_hbm, loss_hbm, cnt_out, sum_out, val_out,
             lbuf, labv, lossv, stage, sem, osem):
    cid = lax.axis_index("c")
    sid = lax.axis_index("s")
    wid = sid * 2 + cid
    img = wid // 4
    quarter = wid % 4
    row0 = quarter * _ROWS_PER_WORK

    zeros = jnp.zeros((16,), jnp.float32)

    def chunk_coords(k):
        stripe = k // _NHALF
        half = k % _NHALF
        return row0 + stripe * _CR, half * _CW

    def fire_chunk(k, slot):
        r0, w0 = chunk_coords(k)
        for c in range(_NC):
            src = logits_hbm.at[img, c, pl.ds(r0, _CR), pl.ds(w0, _CW)]
            pltpu.async_copy(
                src, lbuf.at[pl.ds((slot * _NC + c) * _CR, _CR), :], sem)
        pltpu.async_copy(labels_hbm.at[img, pl.ds(r0, _CR), pl.ds(w0, _CW)],
                         labv.at[pl.ds(slot * _CR, _CR), :], sem)

    def drain_chunk(slot):
        for c in range(_NC):
            pltpu.make_async_copy(
                logits_hbm.at[0, 0, pl.ds(0, _CR), pl.ds(0, _CW)],
                lbuf.at[pl.ds((slot * _NC + c) * _CR, _CR), :], sem).wait()
        pltpu.make_async_copy(
            labels_hbm.at[0, pl.ds(0, _CR), pl.ds(0, _CW)],
            labv.at[pl.ds(slot * _CR, _CR), :], sem).wait()

    def compute_chunk(k, slot, carry):
        def one_group(r, w, acc):
            cnt_h, sum_h, cnt_v = acc
            z = [lbuf[(slot * _NC + c) * _CR + r, pl.ds(w, 16)]
                 for c in range(_NC)]
            m = _tree(z, jnp.maximum)
            s = _tree([jnp.exp(zc - m) for zc in z], jnp.add)
            lab = labv[slot * _CR + r, pl.ds(w, 16)]
            valid = lab != _LB_IGNORE
            labc = jnp.minimum(jnp.maximum(lab, 0), _NC - 1)
            z_l = z[0]
            for c in range(1, _NC):
                z_l = jnp.where(labc == c, z[c], z_l)
            loss = _log_f32(s) - (z_l - m)
            loss = jnp.where(valid, loss, 0.0)
            hard = loss > _THRESH
            cnt_h = cnt_h + jnp.where(hard, 1.0, 0.0)
            sum_h = sum_h + jnp.where(hard, loss, 0.0)
            cnt_v = cnt_v + jnp.where(valid, 1.0, 0.0)
            lossv[slot * _CR + r, pl.ds(w, 16)] = loss
            return (cnt_h, sum_h, cnt_v)

        def pair_group_body(q, acc):
            r = jnp.right_shift(q, 3)
            wb = jnp.bitwise_and(q, 7) * 32
            acc = one_group(r, pl.multiple_of(wb, 16), acc)
            acc = one_group(r, pl.multiple_of(wb + 16, 16), acc)
            return acc

        carry = lax.fori_loop(0, _GROUPS // 2, pair_group_body, carry)
        r0, w0 = chunk_coords(k)
        pltpu.sync_copy(lossv.at[pl.ds(slot * _CR, _CR), :],
                        loss_hbm.at[img, pl.ds(r0, _CR), pl.ds(w0, _CW)])
        return carry

    fire_chunk(0, 0)
    fire_chunk(1, 1)

    def pair_body(j2, carry):
        for slot in (0, 1):
            k = j2 * 2 + slot
            drain_chunk(slot)
            carry = compute_chunk(k, slot, carry)

            @pl.when(k + 2 < _NCHUNK)
            def _():
                fire_chunk(k + 2, slot)
        return carry

    cnt_h, sum_h, cnt_v = lax.fori_loop(
        0, _NCHUNK // 2, pair_body, (zeros, zeros, zeros))

    stage[pl.ds(0, 16)] = cnt_h
    pltpu.sync_copy(stage, cnt_out.at[pl.ds(wid * 16, 16)])
    stage[pl.ds(0, 16)] = sum_h
    pltpu.sync_copy(stage, sum_out.at[pl.ds(wid * 16, 16)])
    stage[pl.ds(0, 16)] = cnt_v
    pltpu.sync_copy(stage, val_out.at[pl.ds(wid * 16, 16)])


def _sc_main(logits, labels):
    mesh = plsc.VectorSubcoreMesh(core_axis_name="c", subcore_axis_name="s")
    f = pl.kernel(
        _sc_body,
        mesh=mesh,
        out_type=[
            jax.ShapeDtypeStruct((_NB, _H, _W), jnp.float32),
            jax.ShapeDtypeStruct((_NWORK * 16,), jnp.float32),
            jax.ShapeDtypeStruct((_NWORK * 16,), jnp.float32),
            jax.ShapeDtypeStruct((_NWORK * 16,), jnp.float32),
        ],
        scratch_types=[
            pltpu.VMEM((2 * _NC * _CR, _CW), jnp.float32),
            pltpu.VMEM((2 * _CR, _CW), jnp.int32),
            pltpu.VMEM((2 * _CR, _CW), jnp.float32),
            pltpu.VMEM((16,), jnp.float32),
            pltpu.SemaphoreType.DMA,
            pltpu.SemaphoreType.DMA,
        ],
    )
    return f(logits, labels)


def _topk_mean_body(x_ref, o_ref):
    x = jnp.maximum(x_ref[...], 0.0)
    bits = lax.bitcast_convert_type(x, jnp.int32)
    kf = jnp.float32(_K_STATIC)

    def step(i, cand):
        test = jnp.bitwise_or(cand, lax.shift_left(jnp.int32(1), 30 - i))
        cnt = jnp.sum((bits >= test).astype(jnp.float32))
        return jnp.where(cnt >= kf, test, cand)

    cand = lax.fori_loop(0, 31, step, jnp.int32(0))
    v = lax.bitcast_convert_type(cand, jnp.float32)
    gt = x > v
    cnt_gt = jnp.sum(gt.astype(jnp.float32))
    sum_gt = jnp.sum(jnp.where(gt, x, 0.0))
    o_ref[0, 0] = (sum_gt + (kf - cnt_gt) * v) / kf


def _topk_mean(loss):
    out = pl.pallas_call(
        _topk_mean_body,
        out_shape=jax.ShapeDtypeStruct((1, 1), jnp.float32),
        out_specs=pl.BlockSpec(memory_space=pltpu.SMEM),
    )(loss)
    return out[0, 0]


def kernel(logits, labels):
    loss, cnt_h, sum_h, cnt_v = _sc_main(logits, labels)
    n_hard = jnp.sum(cnt_h).astype(jnp.int32)
    n_valid = jnp.sum(cnt_v).astype(jnp.int32)
    sum_hard = jnp.sum(sum_h)
    n_min = n_valid // _FACTOR
    mean_hard = sum_hard / jnp.maximum(n_hard, 1).astype(jnp.float32)
    pred = n_hard < n_min
    return lax.cond(pred, _topk_mean, lambda _: mean_hard, loss)


# speedup vs baseline: 9.5385x; 1.0085x over previous
import functools

import jax
import jax.numpy as jnp
from jax import lax
from jax.experimental import pallas as pl
from jax.experimental.pallas import tpu as pltpu
from jax.experimental.pallas import tpu_sc as plsc

_THRESH = 0.35667494393873245
_LB_IGNORE = 255
_FACTOR = 16

_NB, _NC, _H, _W = 8, 19, 512, 512
_NPIX = _NB * _H * _W
_NWORK = 32
_ROWS_PER_WORK = _H // 4
_CR = 8
_CW = 256
_CHUNK = _CR * _CW
_NSTRIPE = _ROWS_PER_WORK // _CR
_NHALF = _W // _CW
_NCHUNK = _NSTRIPE * _NHALF
_GROUPS = _CHUNK // 16
_GROUPS_PER_ROW = _CW // 16

_LN2 = 0.6931471805599453
_K_STATIC = max(_NPIX // _FACTOR, 1)


def _log_f32(s):
    bits = lax.bitcast_convert_type(s, jnp.int32)
    e = jnp.right_shift(bits, 23) - 127
    m = lax.bitcast_convert_type(
        jnp.bitwise_or(jnp.bitwise_and(bits, 0x7FFFFF), 0x3F800000),
        jnp.float32)
    big = m > 1.4142135623730951
    m = jnp.where(big, m * 0.5, m)
    ef = (e + jnp.where(big, 1, 0)).astype(jnp.float32)
    z = m - 1.0
    c8, c7, c6, c5, c4, c3, c2, c1, c0 = (
        7.0376836292e-2, -1.1514610310e-1, 1.1676998740e-1,
        -1.2420140846e-1, 1.4249322787e-1, -1.6668057665e-1,
        2.0000714765e-1, -2.4999993993e-1, 3.3333331174e-1)
    z2 = z * z
    z4 = z2 * z2
    b0 = c1 * z + c0
    b1 = c3 * z + c2
    b2 = c5 * z + c4
    b3 = c7 * z + c6
    d0 = b1 * z2 + b0
    d1 = b3 * z2 + b2
    poly = (c8 * z4 + d1) * z4 + d0
    r = z * z2 * poly - 0.5 * z2
    return z + r + ef * _LN2


def _tree(xs, op):
    xs = list(xs)
    while len(xs) > 1:
        nxt = [op(xs[i], xs[i + 1]) for i in range(0, len(xs) - 1, 2)]
        if len(xs) % 2:
            nxt.append(xs[-1])
        xs = nxt
    return xs[0]


def _sc_body(logits_hbm, labels_hbm, loss_hbm, cnt_out, sum_out, val_out,
             lbuf, labv, lossv, stage, sem, osem):
    cid = lax.axis_index("c")
    sid = lax.axis_index("s")
    wid = sid * 2 + cid
    img = wid // 4
    quarter = wid % 4
    row0 = quarter * _ROWS_PER_WORK

    zeros = jnp.zeros((16,), jnp.float32)

    def chunk_coords(k):
        stripe = k // _NHALF
        half = k % _NHALF
        return row0 + stripe * _CR, half * _CW

    def fire_chunk(k, slot):
        r0, w0 = chunk_coords(k)
        for c in range(_NC):
            src = logits_hbm.at[img, c, pl.ds(r0, _CR), pl.ds(w0, _CW)]
            pltpu.async_copy(
                src, lbuf.at[pl.ds((slot * _NC + c) * _CR, _CR), :], sem)
        pltpu.async_copy(labels_hbm.at[img, pl.ds(r0, _CR), pl.ds(w0, _CW)],
                         labv.at[pl.ds(slot * _CR, _CR), :], sem)

    def drain_chunk(slot):
        for c in range(_NC):
            pltpu.make_async_copy(
                logits_hbm.at[0, 0, pl.ds(0, _CR), pl.ds(0, _CW)],
                lbuf.at[pl.ds((slot * _NC + c) * _CR, _CR), :], sem).wait()
        pltpu.make_async_copy(
            labels_hbm.at[0, pl.ds(0, _CR), pl.ds(0, _CW)],
            labv.at[pl.ds(slot * _CR, _CR), :], sem).wait()

    def compute_chunk(k, slot, carry):
        def one_group(r, w, acc):
            cnt_h, sum_h, cnt_v = acc

            def zload(c):
                return lbuf[(slot * _NC + c) * _CR + r, pl.ds(w, 16)]

            lab = labv[slot * _CR + r, pl.ds(w, 16)]
            valid = lab != _LB_IGNORE
            labc = jnp.minimum(jnp.maximum(lab, 0), _NC - 1)
            z0 = zload(0)
            m = z0
            z_l = z0
            for c in range(1, _NC):
                zc = zload(c)
                m = jnp.maximum(m, zc)
                z_l = jnp.where(labc == c, zc, z_l)
            s = _tree([jnp.exp(zload(c) - m) for c in range(_NC)], jnp.add)
            loss = _log_f32(s) - (z_l - m)
            loss = jnp.where(valid, loss, 0.0)
            hard = loss > _THRESH
            cnt_h = cnt_h + jnp.where(hard, 1.0, 0.0)
            sum_h = sum_h + jnp.where(hard, loss, 0.0)
            cnt_v = cnt_v + jnp.where(valid, 1.0, 0.0)
            lossv[slot * _CR + r, pl.ds(w, 16)] = loss
            return (cnt_h, sum_h, cnt_v)

        def pair_group_body(q, acc):
            r = jnp.right_shift(q, 3)
            wb = jnp.bitwise_and(q, 7) * 32
            acc = one_group(r, pl.multiple_of(wb, 16), acc)
            acc = one_group(r, pl.multiple_of(wb + 16, 16), acc)
            return acc

        carry = lax.fori_loop(0, _GROUPS // 2, pair_group_body, carry)
        r0, w0 = chunk_coords(k)
        pltpu.sync_copy(lossv.at[pl.ds(slot * _CR, _CR), :],
                        loss_hbm.at[img, pl.ds(r0, _CR), pl.ds(w0, _CW)])
        return carry

    fire_chunk(0, 0)
    fire_chunk(1, 1)

    def pair_body(j2, carry):
        for slot in (0, 1):
            k = j2 * 2 + slot
            drain_chunk(slot)
            carry = compute_chunk(k, slot, carry)

            @pl.when(k + 2 < _NCHUNK)
            def _():
                fire_chunk(k + 2, slot)
        return carry

    cnt_h, sum_h, cnt_v = lax.fori_loop(
        0, _NCHUNK // 2, pair_body, (zeros, zeros, zeros))

    stage[pl.ds(0, 16)] = cnt_h
    pltpu.sync_copy(stage, cnt_out.at[pl.ds(wid * 16, 16)])
    stage[pl.ds(0, 16)] = sum_h
    pltpu.sync_copy(stage, sum_out.at[pl.ds(wid * 16, 16)])
    stage[pl.ds(0, 16)] = cnt_v
    pltpu.sync_copy(stage, val_out.at[pl.ds(wid * 16, 16)])


def _sc_main(logits, labels):
    mesh = plsc.VectorSubcoreMesh(core_axis_name="c", subcore_axis_name="s")
    f = pl.kernel(
        _sc_body,
        mesh=mesh,
        out_type=[
            jax.ShapeDtypeStruct((_NB, _H, _W), jnp.float32),
            jax.ShapeDtypeStruct((_NWORK * 16,), jnp.float32),
            jax.ShapeDtypeStruct((_NWORK * 16,), jnp.float32),
            jax.ShapeDtypeStruct((_NWORK * 16,), jnp.float32),
        ],
        scratch_types=[
            pltpu.VMEM((2 * _NC * _CR, _CW), jnp.float32),
            pltpu.VMEM((2 * _CR, _CW), jnp.int32),
            pltpu.VMEM((2 * _CR, _CW), jnp.float32),
            pltpu.VMEM((16,), jnp.float32),
            pltpu.SemaphoreType.DMA,
            pltpu.SemaphoreType.DMA,
        ],
    )
    return f(logits, labels)


def _topk_mean_body(x_ref, o_ref):
    x = jnp.maximum(x_ref[...], 0.0)
    bits = lax.bitcast_convert_type(x, jnp.int32)
    kf = jnp.float32(_K_STATIC)

    def step(i, cand):
        test = jnp.bitwise_or(cand, lax.shift_left(jnp.int32(1), 30 - i))
        cnt = jnp.sum((bits >= test).astype(jnp.float32))
        return jnp.where(cnt >= kf, test, cand)

    cand = lax.fori_loop(0, 31, step, jnp.int32(0))
    v = lax.bitcast_convert_type(cand, jnp.float32)
    gt = x > v
    cnt_gt = jnp.sum(gt.astype(jnp.float32))
    sum_gt = jnp.sum(jnp.where(gt, x, 0.0))
    o_ref[0, 0] = (sum_gt + (kf - cnt_gt) * v) / kf


def _topk_mean(loss):
    out = pl.pallas_call(
        _topk_mean_body,
        out_shape=jax.ShapeDtypeStruct((1, 1), jnp.float32),
        out_specs=pl.BlockSpec(memory_space=pltpu.SMEM),
    )(loss)
    return out[0, 0]


def kernel(logits, labels):
    loss, cnt_h, sum_h, cnt_v = _sc_main(logits, labels)
    n_hard = jnp.sum(cnt_h).astype(jnp.int32)
    n_valid = jnp.sum(cnt_v).astype(jnp.int32)
    sum_hard = jnp.sum(sum_h)
    n_min = n_valid // _FACTOR
    mean_hard = sum_hard / jnp.maximum(n_hard, 1).astype(jnp.float32)
    pred = n_hard < n_min
    return lax.cond(pred, _topk_mean, lambda _: mean_hard, loss)


# speedup vs baseline: 9.6801x; 1.0148x over previous
import functools

import jax
import jax.numpy as jnp
from jax import lax
from jax.experimental import pallas as pl
from jax.experimental.pallas import tpu as pltpu
from jax.experimental.pallas import tpu_sc as plsc

_THRESH = 0.35667494393873245
_LB_IGNORE = 255
_FACTOR = 16

_NB, _NC, _H, _W = 8, 19, 512, 512
_NPIX = _NB * _H * _W
_NWORK = 32
_ROWS_PER_WORK = _H // 4
_CR = 8
_CW = 256
_CHUNK = _CR * _CW
_NSTRIPE = _ROWS_PER_WORK // _CR
_NHALF = _W // _CW
_NCHUNK = _NSTRIPE * _NHALF
_GROUPS = _CHUNK // 16
_GROUPS_PER_ROW = _CW // 16

_LN2 = 0.6931471805599453
_K_STATIC = max(_NPIX // _FACTOR, 1)


def _log_f32(s):
    bits = lax.bitcast_convert_type(s, jnp.int32)
    e = jnp.right_shift(bits, 23) - 127
    m = lax.bitcast_convert_type(
        jnp.bitwise_or(jnp.bitwise_and(bits, 0x7FFFFF), 0x3F800000),
        jnp.float32)
    big = m > 1.4142135623730951
    m = jnp.where(big, m * 0.5, m)
    ef = (e + jnp.where(big, 1, 0)).astype(jnp.float32)
    z = m - 1.0
    c8, c7, c6, c5, c4, c3, c2, c1, c0 = (
        7.0376836292e-2, -1.1514610310e-1, 1.1676998740e-1,
        -1.2420140846e-1, 1.4249322787e-1, -1.6668057665e-1,
        2.0000714765e-1, -2.4999993993e-1, 3.3333331174e-1)
    z2 = z * z
    z4 = z2 * z2
    b0 = c1 * z + c0
    b1 = c3 * z + c2
    b2 = c5 * z + c4
    b3 = c7 * z + c6
    d0 = b1 * z2 + b0
    d1 = b3 * z2 + b2
    poly = (c8 * z4 + d1) * z4 + d0
    r = z * z2 * poly - 0.5 * z2
    return z + r + ef * _LN2


def _tree(xs, op):
    xs = list(xs)
    while len(xs) > 1:
        nxt = [op(xs[i], xs[i + 1]) for i in range(0, len(xs) - 1, 2)]
        if len(xs) % 2:
            nxt.append(xs[-1])
        xs = nxt
    return xs[0]


def _sc_body(logits_hbm, labels_hbm, loss_hbm, cnt_out, sum_out, val_out,
             lbuf, labv, lossv, stage, sem, osem):
    cid = lax.axis_index("c")
    sid = lax.axis_index("s")
    wid = sid * 2 + cid
    img = wid // 4
    quarter = wid % 4
    row0 = quarter * _ROWS_PER_WORK

    zeros = jnp.zeros((16,), jnp.float32)

    def chunk_coords(k):
        stripe = k // _NHALF
        half = k % _NHALF
        return row0 + stripe * _CR, half * _CW

    def fire_chunk(k, slot):
        r0, w0 = chunk_coords(k)
        for c in range(_NC):
            src = logits_hbm.at[img, c, pl.ds(r0, _CR), pl.ds(w0, _CW)]
            pltpu.async_copy(
                src, lbuf.at[pl.ds((slot * _NC + c) * _CR, _CR), :], sem)
        pltpu.async_copy(labels_hbm.at[img, pl.ds(r0, _CR), pl.ds(w0, _CW)],
                         labv.at[pl.ds(slot * _CR, _CR), :], sem)

    def drain_chunk(slot):
        for c in range(_NC):
            pltpu.make_async_copy(
                logits_hbm.at[0, 0, pl.ds(0, _CR), pl.ds(0, _CW)],
                lbuf.at[pl.ds((slot * _NC + c) * _CR, _CR), :], sem).wait()
        pltpu.make_async_copy(
            labels_hbm.at[0, pl.ds(0, _CR), pl.ds(0, _CW)],
            labv.at[pl.ds(slot * _CR, _CR), :], sem).wait()

    def compute_chunk(k, slot, carry):
        def one_group(r, w, acc):
            cnt_h, sum_h, cnt_v = acc

            def zload(c):
                return lbuf[(slot * _NC + c) * _CR + r, pl.ds(w, 16)]

            lab = labv[slot * _CR + r, pl.ds(w, 16)]
            valid = lab != _LB_IGNORE
            labc = jnp.minimum(jnp.maximum(lab, 0), _NC - 1)
            z0 = zload(0)
            m = z0
            z_l = z0
            for c in range(1, _NC):
                zc = zload(c)
                m = jnp.maximum(m, zc)
                z_l = jnp.where(labc == c, zc, z_l)
            s = _tree([jnp.exp(zload(c) - m) for c in range(_NC)], jnp.add)
            loss = _log_f32(s) - (z_l - m)
            loss = jnp.where(valid, loss, 0.0)
            hard = loss > _THRESH
            cnt_h = cnt_h + jnp.where(hard, 1.0, 0.0)
            sum_h = sum_h + jnp.where(hard, loss, 0.0)
            cnt_v = cnt_v + jnp.where(valid, 1.0, 0.0)
            lossv[slot * _CR + r, pl.ds(w, 16)] = loss
            return (cnt_h, sum_h, cnt_v)

        def pair_group_body(q, acc):
            r = jnp.right_shift(q, 3)
            wb = jnp.bitwise_and(q, 7) * 32
            acc = one_group(r, pl.multiple_of(wb, 16), acc)
            acc = one_group(r, pl.multiple_of(wb + 16, 16), acc)
            return acc

        carry = lax.fori_loop(0, _GROUPS // 2, pair_group_body, carry)
        r0, w0 = chunk_coords(k)
        pltpu.async_copy(lossv.at[pl.ds(slot * _CR, _CR), :],
                         loss_hbm.at[img, pl.ds(r0, _CR), pl.ds(w0, _CW)],
                         osem)
        return carry

    def drain_loss(slot):
        pltpu.make_async_copy(
            lossv.at[pl.ds(slot * _CR, _CR), :],
            loss_hbm.at[0, pl.ds(0, _CR), pl.ds(0, _CW)], osem).wait()

    fire_chunk(0, 0)
    fire_chunk(1, 1)

    def pair_body(j2, carry):
        for slot in (0, 1):
            k = j2 * 2 + slot
            drain_chunk(slot)

            @pl.when(k >= 2)
            def _():
                drain_loss(slot)

            carry = compute_chunk(k, slot, carry)

            @pl.when(k + 2 < _NCHUNK)
            def _():
                fire_chunk(k + 2, slot)
        return carry

    cnt_h, sum_h, cnt_v = lax.fori_loop(
        0, _NCHUNK // 2, pair_body, (zeros, zeros, zeros))
    drain_loss(0)
    drain_loss(1)

    stage[pl.ds(0, 16)] = cnt_h
    pltpu.sync_copy(stage, cnt_out.at[pl.ds(wid * 16, 16)])
    stage[pl.ds(0, 16)] = sum_h
    pltpu.sync_copy(stage, sum_out.at[pl.ds(wid * 16, 16)])
    stage[pl.ds(0, 16)] = cnt_v
    pltpu.sync_copy(stage, val_out.at[pl.ds(wid * 16, 16)])


def _sc_main(logits, labels):
    mesh = plsc.VectorSubcoreMesh(core_axis_name="c", subcore_axis_name="s")
    f = pl.kernel(
        _sc_body,
        mesh=mesh,
        out_type=[
            jax.ShapeDtypeStruct((_NB, _H, _W), jnp.float32),
            jax.ShapeDtypeStruct((_NWORK * 16,), jnp.float32),
            jax.ShapeDtypeStruct((_NWORK * 16,), jnp.float32),
            jax.ShapeDtypeStruct((_NWORK * 16,), jnp.float32),
        ],
        scratch_types=[
            pltpu.VMEM((2 * _NC * _CR, _CW), jnp.float32),
            pltpu.VMEM((2 * _CR, _CW), jnp.int32),
            pltpu.VMEM((2 * _CR, _CW), jnp.float32),
            pltpu.VMEM((16,), jnp.float32),
            pltpu.SemaphoreType.DMA,
            pltpu.SemaphoreType.DMA,
        ],
    )
    return f(logits, labels)


def _topk_mean_body(x_ref, o_ref):
    x = jnp.maximum(x_ref[...], 0.0)
    bits = lax.bitcast_convert_type(x, jnp.int32)
    kf = jnp.float32(_K_STATIC)

    def step(i, cand):
        test = jnp.bitwise_or(cand, lax.shift_left(jnp.int32(1), 30 - i))
        cnt = jnp.sum((bits >= test).astype(jnp.float32))
        return jnp.where(cnt >= kf, test, cand)

    cand = lax.fori_loop(0, 31, step, jnp.int32(0))
    v = lax.bitcast_convert_type(cand, jnp.float32)
    gt = x > v
    cnt_gt = jnp.sum(gt.astype(jnp.float32))
    sum_gt = jnp.sum(jnp.where(gt, x, 0.0))
    o_ref[0, 0] = (sum_gt + (kf - cnt_gt) * v) / kf


def _topk_mean(loss):
    out = pl.pallas_call(
        _topk_mean_body,
        out_shape=jax.ShapeDtypeStruct((1, 1), jnp.float32),
        out_specs=pl.BlockSpec(memory_space=pltpu.SMEM),
    )(loss)
    return out[0, 0]


def kernel(logits, labels):
    loss, cnt_h, sum_h, cnt_v = _sc_main(logits, labels)
    n_hard = jnp.sum(cnt_h).astype(jnp.int32)
    n_valid = jnp.sum(cnt_v).astype(jnp.int32)
    sum_hard = jnp.sum(sum_h)
    n_min = n_valid // _FACTOR
    mean_hard = sum_hard / jnp.maximum(n_hard, 1).astype(jnp.float32)
    pred = n_hard < n_min
    return lax.cond(pred, _topk_mean, lambda _: mean_hard, loss)


# speedup vs baseline: 24.5232x; 2.5334x over previous
import functools

import jax
import jax.numpy as jnp
from jax import lax
from jax.experimental import pallas as pl
from jax.experimental.pallas import tpu as pltpu
from jax.experimental.pallas import tpu_sc as plsc

_THRESH = 0.35667494393873245
_LB_IGNORE = 255
_FACTOR = 16

_NB, _NC, _H, _W = 8, 19, 512, 512
_NPIX = _NB * _H * _W
_K_STATIC = max(_NPIX // _FACTOR, 1)
_LN2 = 0.6931471805599453

_SCI = 2
_TCI = _NB - _SCI

_NWORK = 32
_WPI = _NWORK // _SCI
_ROWS_PER_WORK = _H // _WPI
_CR = 8
_CW = 256
_CHUNK = _CR * _CW
_NSTRIPE = _ROWS_PER_WORK // _CR
_NHALF = _W // _CW
_NCHUNK = _NSTRIPE * _NHALF
_GROUPS = _CHUNK // 16

_TC_RB = 64
_TC_NRB = _H // _TC_RB
_TC_STEPS = _TCI * _TC_NRB


def _log_f32(s):
    bits = lax.bitcast_convert_type(s, jnp.int32)
    e = jnp.right_shift(bits, 23) - 127
    m = lax.bitcast_convert_type(
        jnp.bitwise_or(jnp.bitwise_and(bits, 0x7FFFFF), 0x3F800000),
        jnp.float32)
    big = m > 1.4142135623730951
    m = jnp.where(big, m * 0.5, m)
    ef = (e + jnp.where(big, 1, 0)).astype(jnp.float32)
    z = m - 1.0
    c8, c7, c6, c5, c4, c3, c2, c1, c0 = (
        7.0376836292e-2, -1.1514610310e-1, 1.1676998740e-1,
        -1.2420140846e-1, 1.4249322787e-1, -1.6668057665e-1,
        2.0000714765e-1, -2.4999993993e-1, 3.3333331174e-1)
    z2 = z * z
    z4 = z2 * z2
    b0 = c1 * z + c0
    b1 = c3 * z + c2
    b2 = c5 * z + c4
    b3 = c7 * z + c6
    d0 = b1 * z2 + b0
    d1 = b3 * z2 + b2
    poly = (c8 * z4 + d1) * z4 + d0
    r = z * z2 * poly - 0.5 * z2
    return z + r + ef * _LN2


def _tree(xs, op):
    xs = list(xs)
    while len(xs) > 1:
        nxt = [op(xs[i], xs[i + 1]) for i in range(0, len(xs) - 1, 2)]
        if len(xs) % 2:
            nxt.append(xs[-1])
        xs = nxt
    return xs[0]


def _sc_body(logits_hbm, labels_hbm, loss_hbm, cnt_out, sum_out, val_out,
             lbuf, labv, lossv, stage, sem, osem):
    cid = lax.axis_index("c")
    sid = lax.axis_index("s")
    wid = sid * 2 + cid
    img = wid // _WPI
    row0 = (wid % _WPI) * _ROWS_PER_WORK

    zeros = jnp.zeros((16,), jnp.float32)

    def chunk_coords(k):
        stripe = k // _NHALF
        half = k % _NHALF
        return row0 + stripe * _CR, half * _CW

    def fire_chunk(k, slot):
        r0, w0 = chunk_coords(k)
        for c in range(_NC):
            src = logits_hbm.at[img, c, pl.ds(r0, _CR), pl.ds(w0, _CW)]
            pltpu.async_copy(
                src, lbuf.at[pl.ds((slot * _NC + c) * _CR, _CR), :], sem)
        pltpu.async_copy(labels_hbm.at[img, pl.ds(r0, _CR), pl.ds(w0, _CW)],
                         labv.at[pl.ds(slot * _CR, _CR), :], sem)

    def drain_chunk(slot):
        for c in range(_NC):
            pltpu.make_async_copy(
                logits_hbm.at[0, 0, pl.ds(0, _CR), pl.ds(0, _CW)],
                lbuf.at[pl.ds((slot * _NC + c) * _CR, _CR), :], sem).wait()
        pltpu.make_async_copy(
            labels_hbm.at[0, pl.ds(0, _CR), pl.ds(0, _CW)],
            labv.at[pl.ds(slot * _CR, _CR), :], sem).wait()

    def compute_chunk(k, slot, carry):
        def one_group(r, w, acc):
            cnt_h, sum_h, cnt_v = acc

            def zload(c):
                return lbuf[(slot * _NC + c) * _CR + r, pl.ds(w, 16)]

            lab = labv[slot * _CR + r, pl.ds(w, 16)]
            valid = lab != _LB_IGNORE
            labc = jnp.minimum(jnp.maximum(lab, 0), _NC - 1)
            z0 = zload(0)
            m = z0
            z_l = z0
            for c in range(1, _NC):
                zc = zload(c)
                m = jnp.maximum(m, zc)
                z_l = jnp.where(labc == c, zc, z_l)
            s = _tree([jnp.exp(zload(c) - m) for c in range(_NC)], jnp.add)
            loss = _log_f32(s) - (z_l - m)
            loss = jnp.where(valid, loss, 0.0)
            hard = loss > _THRESH
            cnt_h = cnt_h + jnp.where(hard, 1.0, 0.0)
            sum_h = sum_h + jnp.where(hard, loss, 0.0)
            cnt_v = cnt_v + jnp.where(valid, 1.0, 0.0)
            lossv[slot * _CR + r, pl.ds(w, 16)] = loss
            return (cnt_h, sum_h, cnt_v)

        def pair_group_body(q, acc):
            r = jnp.right_shift(q, 3)
            wb = jnp.bitwise_and(q, 7) * 32
            acc = one_group(r, pl.multiple_of(wb, 16), acc)
            acc = one_group(r, pl.multiple_of(wb + 16, 16), acc)
            return acc

        carry = lax.fori_loop(0, _GROUPS // 2, pair_group_body, carry)
        r0, w0 = chunk_coords(k)
        pltpu.async_copy(lossv.at[pl.ds(slot * _CR, _CR), :],
                         loss_hbm.at[img, pl.ds(r0, _CR), pl.ds(w0, _CW)],
                         osem)
        return carry

    def drain_loss(slot):
        pltpu.make_async_copy(
            lossv.at[pl.ds(slot * _CR, _CR), :],
            loss_hbm.at[0, pl.ds(0, _CR), pl.ds(0, _CW)], osem).wait()

    fire_chunk(0, 0)
    fire_chunk(1, 1)

    def pair_body(j2, carry):
        for slot in (0, 1):
            k = j2 * 2 + slot
            drain_chunk(slot)

            @pl.when(k >= 2)
            def _():
                drain_loss(slot)

            carry = compute_chunk(k, slot, carry)

            @pl.when(k + 2 < _NCHUNK)
            def _():
                fire_chunk(k + 2, slot)
        return carry

    cnt_h, sum_h, cnt_v = lax.fori_loop(
        0, _NCHUNK // 2, pair_body, (zeros, zeros, zeros))
    drain_loss(0)
    drain_loss(1)

    stage[pl.ds(0, 16)] = cnt_h
    pltpu.sync_copy(stage, cnt_out.at[pl.ds(wid * 16, 16)])
    stage[pl.ds(0, 16)] = sum_h
    pltpu.sync_copy(stage, sum_out.at[pl.ds(wid * 16, 16)])
    stage[pl.ds(0, 16)] = cnt_v
    pltpu.sync_copy(stage, val_out.at[pl.ds(wid * 16, 16)])


def _sc_main(logits, labels):
    mesh = plsc.VectorSubcoreMesh(core_axis_name="c", subcore_axis_name="s")
    f = pl.kernel(
        _sc_body,
        mesh=mesh,
        out_type=[
            jax.ShapeDtypeStruct((_SCI, _H, _W), jnp.float32),
            jax.ShapeDtypeStruct((_NWORK * 16,), jnp.float32),
            jax.ShapeDtypeStruct((_NWORK * 16,), jnp.float32),
            jax.ShapeDtypeStruct((_NWORK * 16,), jnp.float32),
        ],
        scratch_types=[
            pltpu.VMEM((2 * _NC * _CR, _CW), jnp.float32),
            pltpu.VMEM((2 * _CR, _CW), jnp.int32),
            pltpu.VMEM((2 * _CR, _CW), jnp.float32),
            pltpu.VMEM((16,), jnp.float32),
            pltpu.SemaphoreType.DMA,
            pltpu.SemaphoreType.DMA,
        ],
    )
    return f(logits, labels)


def _tc_body(x_ref, lab_ref, loss_ref, part_ref):
    x = x_ref[0]
    lab = lab_ref[0]
    m = jnp.max(x, axis=0)
    s = jnp.sum(jnp.exp(x - m[None]), axis=0)
    valid = lab != _LB_IGNORE
    labc = jnp.minimum(jnp.maximum(lab, 0), _NC - 1)
    z_l = x[0]
    for c in range(1, _NC):
        z_l = jnp.where(labc == c, x[c], z_l)
    loss = jnp.log(s) - (z_l - m)
    loss = jnp.where(valid, loss, 0.0)
    loss_ref[0] = loss
    hard = loss > _THRESH
    part_ref[0, 0, 0] = jnp.sum(hard.astype(jnp.float32))
    part_ref[0, 0, 1] = jnp.sum(jnp.where(hard, loss, 0.0))
    part_ref[0, 0, 2] = jnp.sum(valid.astype(jnp.float32))


def _tc_main(logits, labels):
    return pl.pallas_call(
        _tc_body,
        grid=(_TCI, _TC_NRB),
        in_specs=[
            pl.BlockSpec((1, _NC, _TC_RB, _W),
                         lambda j, i: (j + _SCI, 0, i, 0)),
            pl.BlockSpec((1, _TC_RB, _W), lambda j, i: (j + _SCI, i, 0)),
        ],
        out_specs=[
            pl.BlockSpec((1, _TC_RB, _W), lambda j, i: (j, i, 0)),
            pl.BlockSpec((1, 1, 3), lambda j, i: (j * _TC_NRB + i, 0, 0),
                         memory_space=pltpu.SMEM),
        ],
        out_shape=[
            jax.ShapeDtypeStruct((_TCI, _H, _W), jnp.float32),
            jax.ShapeDtypeStruct((_TC_STEPS, 1, 3), jnp.float32),
        ],
    )(logits, labels)


def _topk_mean_body(x1_ref, x2_ref, o_ref):
    x1 = jnp.maximum(x1_ref[...], 0.0)
    x2 = jnp.maximum(x2_ref[...], 0.0)
    b1 = lax.bitcast_convert_type(x1, jnp.int32)
    b2 = lax.bitcast_convert_type(x2, jnp.int32)
    kf = jnp.float32(_K_STATIC)

    def step(i, cand):
        test = jnp.bitwise_or(cand, lax.shift_left(jnp.int32(1), 30 - i))
        cnt = (jnp.sum((b1 >= test).astype(jnp.float32))
               + jnp.sum((b2 >= test).astype(jnp.float32)))
        return jnp.where(cnt >= kf, test, cand)

    cand = lax.fori_loop(0, 31, step, jnp.int32(0))
    v = lax.bitcast_convert_type(cand, jnp.float32)
    g1 = x1 > v
    g2 = x2 > v
    cnt_gt = (jnp.sum(g1.astype(jnp.float32))
              + jnp.sum(g2.astype(jnp.float32)))
    sum_gt = (jnp.sum(jnp.where(g1, x1, 0.0))
              + jnp.sum(jnp.where(g2, x2, 0.0)))
    o_ref[0, 0] = (sum_gt + (kf - cnt_gt) * v) / kf


def _topk_mean(losses):
    loss_sc, loss_tc = losses
    out = pl.pallas_call(
        _topk_mean_body,
        out_shape=jax.ShapeDtypeStruct((1, 1), jnp.float32),
        out_specs=pl.BlockSpec(memory_space=pltpu.SMEM),
    )(loss_sc, loss_tc)
    return out[0, 0]


def kernel(logits, labels):
    loss_sc, cnt_h, sum_h, cnt_v = _sc_main(logits, labels)
    loss_tc, tc_part = _tc_main(logits, labels)
    n_hard_f = jnp.sum(cnt_h) + jnp.sum(tc_part[:, 0, 0])
    sum_hard = jnp.sum(sum_h) + jnp.sum(tc_part[:, 0, 1])
    n_valid_f = jnp.sum(cnt_v) + jnp.sum(tc_part[:, 0, 2])
    n_hard = n_hard_f.astype(jnp.int32)
    n_min = n_valid_f.astype(jnp.int32) // _FACTOR
    mean_hard = sum_hard / jnp.maximum(n_hard, 1).astype(jnp.float32)
    pred = n_hard < n_min
    return lax.cond(pred, _topk_mean, lambda _: mean_hard,
                    (loss_sc, loss_tc))


# speedup vs baseline: 26.3181x; 1.0732x over previous
import functools

import jax
import jax.numpy as jnp
from jax import lax
from jax.experimental import pallas as pl
from jax.experimental.pallas import tpu as pltpu
from jax.experimental.pallas import tpu_sc as plsc

_THRESH = 0.35667494393873245
_LB_IGNORE = 255
_FACTOR = 16

_NB, _NC, _H, _W = 8, 19, 512, 512
_NPIX = _NB * _H * _W
_K_STATIC = max(_NPIX // _FACTOR, 1)
_LN2 = 0.6931471805599453

_SCI = 2
_TCI = _NB - _SCI

_NWORK = 32
_WPI = _NWORK // _SCI
_ROWS_PER_WORK = _H // _WPI
_CR = 8
_CW = 256
_CHUNK = _CR * _CW
_NSTRIPE = _ROWS_PER_WORK // _CR
_NHALF = _W // _CW
_NCHUNK = _NSTRIPE * _NHALF
_GROUPS = _CHUNK // 16

_TC_RB = 128
_TC_NRB = _H // _TC_RB
_TC_STEPS = _TCI * _TC_NRB


def _log_f32(s):
    bits = lax.bitcast_convert_type(s, jnp.int32)
    e = jnp.right_shift(bits, 23) - 127
    m = lax.bitcast_convert_type(
        jnp.bitwise_or(jnp.bitwise_and(bits, 0x7FFFFF), 0x3F800000),
        jnp.float32)
    big = m > 1.4142135623730951
    m = jnp.where(big, m * 0.5, m)
    ef = (e + jnp.where(big, 1, 0)).astype(jnp.float32)
    z = m - 1.0
    c8, c7, c6, c5, c4, c3, c2, c1, c0 = (
        7.0376836292e-2, -1.1514610310e-1, 1.1676998740e-1,
        -1.2420140846e-1, 1.4249322787e-1, -1.6668057665e-1,
        2.0000714765e-1, -2.4999993993e-1, 3.3333331174e-1)
    z2 = z * z
    z4 = z2 * z2
    b0 = c1 * z + c0
    b1 = c3 * z + c2
    b2 = c5 * z + c4
    b3 = c7 * z + c6
    d0 = b1 * z2 + b0
    d1 = b3 * z2 + b2
    poly = (c8 * z4 + d1) * z4 + d0
    r = z * z2 * poly - 0.5 * z2
    return z + r + ef * _LN2


def _tree(xs, op):
    xs = list(xs)
    while len(xs) > 1:
        nxt = [op(xs[i], xs[i + 1]) for i in range(0, len(xs) - 1, 2)]
        if len(xs) % 2:
            nxt.append(xs[-1])
        xs = nxt
    return xs[0]


def _sc_body(logits_hbm, labels_hbm, loss_hbm, cnt_out, sum_out, val_out,
             lbuf, labv, lossv, stage, sem, osem):
    cid = lax.axis_index("c")
    sid = lax.axis_index("s")
    wid = sid * 2 + cid
    img = wid // _WPI
    row0 = (wid % _WPI) * _ROWS_PER_WORK

    zeros = jnp.zeros((16,), jnp.float32)

    def chunk_coords(k):
        stripe = k // _NHALF
        half = k % _NHALF
        return row0 + stripe * _CR, half * _CW

    def fire_chunk(k, slot):
        r0, w0 = chunk_coords(k)
        for c in range(_NC):
            src = logits_hbm.at[img, c, pl.ds(r0, _CR), pl.ds(w0, _CW)]
            pltpu.async_copy(
                src, lbuf.at[pl.ds((slot * _NC + c) * _CR, _CR), :], sem)
        pltpu.async_copy(labels_hbm.at[img, pl.ds(r0, _CR), pl.ds(w0, _CW)],
                         labv.at[pl.ds(slot * _CR, _CR), :], sem)

    def drain_chunk(slot):
        for c in range(_NC):
            pltpu.make_async_copy(
                logits_hbm.at[0, 0, pl.ds(0, _CR), pl.ds(0, _CW)],
                lbuf.at[pl.ds((slot * _NC + c) * _CR, _CR), :], sem).wait()
        pltpu.make_async_copy(
            labels_hbm.at[0, pl.ds(0, _CR), pl.ds(0, _CW)],
            labv.at[pl.ds(slot * _CR, _CR), :], sem).wait()

    def compute_chunk(k, slot, carry):
        def one_group(r, w, acc):
            cnt_h, sum_h, cnt_v = acc

            def zload(c):
                return lbuf[(slot * _NC + c) * _CR + r, pl.ds(w, 16)]

            lab = labv[slot * _CR + r, pl.ds(w, 16)]
            valid = lab != _LB_IGNORE
            labc = jnp.minimum(jnp.maximum(lab, 0), _NC - 1)
            z0 = zload(0)
            m = z0
            z_l = z0
            for c in range(1, _NC):
                zc = zload(c)
                m = jnp.maximum(m, zc)
                z_l = jnp.where(labc == c, zc, z_l)
            s = _tree([jnp.exp(zload(c) - m) for c in range(_NC)], jnp.add)
            loss = _log_f32(s) - (z_l - m)
            loss = jnp.where(valid, loss, 0.0)
            hard = loss > _THRESH
            cnt_h = cnt_h + jnp.where(hard, 1.0, 0.0)
            sum_h = sum_h + jnp.where(hard, loss, 0.0)
            cnt_v = cnt_v + jnp.where(valid, 1.0, 0.0)
            lossv[slot * _CR + r, pl.ds(w, 16)] = loss
            return (cnt_h, sum_h, cnt_v)

        def pair_group_body(q, acc):
            r = jnp.right_shift(q, 3)
            wb = jnp.bitwise_and(q, 7) * 32
            acc = one_group(r, pl.multiple_of(wb, 16), acc)
            acc = one_group(r, pl.multiple_of(wb + 16, 16), acc)
            return acc

        carry = lax.fori_loop(0, _GROUPS // 2, pair_group_body, carry)
        r0, w0 = chunk_coords(k)
        pltpu.async_copy(lossv.at[pl.ds(slot * _CR, _CR), :],
                         loss_hbm.at[img, pl.ds(r0, _CR), pl.ds(w0, _CW)],
                         osem)
        return carry

    def drain_loss(slot):
        pltpu.make_async_copy(
            lossv.at[pl.ds(slot * _CR, _CR), :],
            loss_hbm.at[0, pl.ds(0, _CR), pl.ds(0, _CW)], osem).wait()

    fire_chunk(0, 0)
    fire_chunk(1, 1)

    def pair_body(j2, carry):
        for slot in (0, 1):
            k = j2 * 2 + slot
            drain_chunk(slot)

            @pl.when(k >= 2)
            def _():
                drain_loss(slot)

            carry = compute_chunk(k, slot, carry)

            @pl.when(k + 2 < _NCHUNK)
            def _():
                fire_chunk(k + 2, slot)
        return carry

    cnt_h, sum_h, cnt_v = lax.fori_loop(
        0, _NCHUNK // 2, pair_body, (zeros, zeros, zeros))
    drain_loss(0)
    drain_loss(1)

    stage[pl.ds(0, 16)] = cnt_h
    pltpu.sync_copy(stage, cnt_out.at[pl.ds(wid * 16, 16)])
    stage[pl.ds(0, 16)] = sum_h
    pltpu.sync_copy(stage, sum_out.at[pl.ds(wid * 16, 16)])
    stage[pl.ds(0, 16)] = cnt_v
    pltpu.sync_copy(stage, val_out.at[pl.ds(wid * 16, 16)])


def _sc_main(logits, labels):
    mesh = plsc.VectorSubcoreMesh(core_axis_name="c", subcore_axis_name="s")
    f = pl.kernel(
        _sc_body,
        mesh=mesh,
        out_type=[
            jax.ShapeDtypeStruct((_SCI, _H, _W), jnp.float32),
            jax.ShapeDtypeStruct((_NWORK * 16,), jnp.float32),
            jax.ShapeDtypeStruct((_NWORK * 16,), jnp.float32),
            jax.ShapeDtypeStruct((_NWORK * 16,), jnp.float32),
        ],
        scratch_types=[
            pltpu.VMEM((2 * _NC * _CR, _CW), jnp.float32),
            pltpu.VMEM((2 * _CR, _CW), jnp.int32),
            pltpu.VMEM((2 * _CR, _CW), jnp.float32),
            pltpu.VMEM((16,), jnp.float32),
            pltpu.SemaphoreType.DMA,
            pltpu.SemaphoreType.DMA,
        ],
    )
    return f(logits, labels)


def _tc_body(x_ref, lab_ref, loss_ref, part_ref):
    x = x_ref[0]
    lab = lab_ref[0]
    m = jnp.max(x, axis=0)
    s = jnp.sum(jnp.exp(x - m[None]), axis=0)
    valid = lab != _LB_IGNORE
    labc = jnp.minimum(jnp.maximum(lab, 0), _NC - 1)
    z_l = x[0]
    for c in range(1, _NC):
        z_l = jnp.where(labc == c, x[c], z_l)
    loss = jnp.log(s) - (z_l - m)
    loss = jnp.where(valid, loss, 0.0)
    loss_ref[0] = loss
    hard = loss > _THRESH
    part_ref[0, 0, 0] = jnp.sum(hard.astype(jnp.float32))
    part_ref[0, 0, 1] = jnp.sum(jnp.where(hard, loss, 0.0))
    part_ref[0, 0, 2] = jnp.sum(valid.astype(jnp.float32))


def _tc_main(logits, labels):
    return pl.pallas_call(
        _tc_body,
        grid=(_TCI, _TC_NRB),
        in_specs=[
            pl.BlockSpec((1, _NC, _TC_RB, _W),
                         lambda j, i: (j + _SCI, 0, i, 0)),
            pl.BlockSpec((1, _TC_RB, _W), lambda j, i: (j + _SCI, i, 0)),
        ],
        out_specs=[
            pl.BlockSpec((1, _TC_RB, _W), lambda j, i: (j, i, 0)),
            pl.BlockSpec((1, 1, 3), lambda j, i: (j * _TC_NRB + i, 0, 0),
                         memory_space=pltpu.SMEM),
        ],
        out_shape=[
            jax.ShapeDtypeStruct((_TCI, _H, _W), jnp.float32),
            jax.ShapeDtypeStruct((_TC_STEPS, 1, 3), jnp.float32),
        ],
    )(logits, labels)


def _topk_mean_body(x1_ref, x2_ref, o_ref):
    x1 = jnp.maximum(x1_ref[...], 0.0)
    x2 = jnp.maximum(x2_ref[...], 0.0)
    b1 = lax.bitcast_convert_type(x1, jnp.int32)
    b2 = lax.bitcast_convert_type(x2, jnp.int32)
    kf = jnp.float32(_K_STATIC)

    def step(i, cand):
        test = jnp.bitwise_or(cand, lax.shift_left(jnp.int32(1), 30 - i))
        cnt = (jnp.sum((b1 >= test).astype(jnp.float32))
               + jnp.sum((b2 >= test).astype(jnp.float32)))
        return jnp.where(cnt >= kf, test, cand)

    cand = lax.fori_loop(0, 31, step, jnp.int32(0))
    v = lax.bitcast_convert_type(cand, jnp.float32)
    g1 = x1 > v
    g2 = x2 > v
    cnt_gt = (jnp.sum(g1.astype(jnp.float32))
              + jnp.sum(g2.astype(jnp.float32)))
    sum_gt = (jnp.sum(jnp.where(g1, x1, 0.0))
              + jnp.sum(jnp.where(g2, x2, 0.0)))
    o_ref[0, 0] = (sum_gt + (kf - cnt_gt) * v) / kf


def _topk_mean(losses):
    loss_sc, loss_tc = losses
    out = pl.pallas_call(
        _topk_mean_body,
        out_shape=jax.ShapeDtypeStruct((1, 1), jnp.float32),
        out_specs=pl.BlockSpec(memory_space=pltpu.SMEM),
    )(loss_sc, loss_tc)
    return out[0, 0]


def kernel(logits, labels):
    loss_sc, cnt_h, sum_h, cnt_v = _sc_main(logits, labels)
    loss_tc, tc_part = _tc_main(logits, labels)
    n_hard_f = jnp.sum(cnt_h) + jnp.sum(tc_part[:, 0, 0])
    sum_hard = jnp.sum(sum_h) + jnp.sum(tc_part[:, 0, 1])
    n_valid_f = jnp.sum(cnt_v) + jnp.sum(tc_part[:, 0, 2])
    n_hard = n_hard_f.astype(jnp.int32)
    n_min = n_valid_f.astype(jnp.int32) // _FACTOR
    mean_hard = sum_hard / jnp.maximum(n_hard, 1).astype(jnp.float32)
    pred = n_hard < n_min
    return lax.cond(pred, _topk_mean, lambda _: mean_hard,
                    (loss_sc, loss_tc))
